# baseline (speedup 1.0000x reference)
import functools

import jax
import jax.numpy as jnp
from jax.experimental import pallas as pl
from jax.experimental.pallas import tpu as pltpu

_B = 1048576
_D = 16
_K = 64
_BB = 8192
_R = _BB // 8
_GCOL = 136


def _vq_block(x_ref, ct2_ref, c2_ref, g3_ref, zq_ref, idx_ref, msum_ref):
    x = x_ref[...]
    e = jax.lax.dot_general(x, ct2_ref[...], (((1,), (0,)), ((), ())),
                            preferred_element_type=jnp.float32)
    e = e + c2_ref[...]
    m = jnp.minimum(e[:, :256], e[:, 256:])
    m = jnp.minimum(m[:, :128], m[:, 128:])
    for s in (8, 16, 32, 64):
        m = jnp.minimum(m, pltpu.roll(m, s, 1))
    mbig = jnp.concatenate([m, m, m, m], axis=1)
    onehot = (e == mbig).astype(jnp.float32)
    out2 = jax.lax.dot_general(onehot, g3_ref[...], (((1,), (0,)), ((), ())),
                               preferred_element_type=jnp.float32)
    zq_ref[...] = out2[:, :128]
    idx_ref[...] = out2[:, 128:_GCOL].astype(jnp.int32)
    s_val = jnp.sum(x * x) + jnp.sum(m) * (1.0 / 16.0)

    @pl.when(pl.program_id(0) == 0)
    def _init():
        msum_ref[0, 0] = 0.0

    msum_ref[0, 0] += s_val


def _build_tables(codebook):
    l = jnp.arange(8 * _K)
    k_of_l = l // 8
    g_of_l = l % 8
    row = jnp.arange(8 * _D)[:, None]
    d = row - _D * g_of_l[None, :]
    ct2 = jnp.where((d >= 0) & (d < _D),
                    -2.0 * codebook[k_of_l[None, :], jnp.clip(d, 0, _D - 1)],
                    0.0)
    c2 = jnp.sum(codebook * codebook, axis=1)[k_of_l][None, :]

    cc = jnp.arange(_GCOL)[None, :]
    lc = l[:, None]
    zpart = jnp.where((cc < 128) & ((lc % 8) == (cc // _D)),
                      codebook[lc // 8, jnp.clip(cc % _D, 0, _D - 1)], 0.0)
    ipart = jnp.where((cc >= 128) & ((lc % 8) == (cc - 128)),
                      (lc // 8).astype(jnp.float32), 0.0)
    g3 = zpart + ipart
    return ct2, c2, g3


def kernel(action, codebook):
    xv = action.reshape(_B // 8, 128)
    ct2, c2, g3 = _build_tables(codebook)
    n_blocks = _B // _BB
    zq, idx, msum = pl.pallas_call(
        _vq_block,
        grid=(n_blocks,),
        in_specs=[
            pl.BlockSpec((_R, 128), lambda i: (i, 0)),
            pl.BlockSpec((128, 8 * _K), lambda i: (0, 0)),
            pl.BlockSpec((1, 8 * _K), lambda i: (0, 0)),
            pl.BlockSpec((8 * _K, _GCOL), lambda i: (0, 0)),
        ],
        out_specs=[
            pl.BlockSpec((_R, 128), lambda i: (i, 0)),
            pl.BlockSpec((_R, 8), lambda i: (i, 0)),
            pl.BlockSpec((1, 1), lambda i: (0, 0), memory_space=pltpu.SMEM),
        ],
        out_shape=[
            jax.ShapeDtypeStruct((_B // 8, 128), jnp.float32),
            jax.ShapeDtypeStruct((_B // 8, 8), jnp.int32),
            jax.ShapeDtypeStruct((1, 1), jnp.float32),
        ],
    )(xv, ct2, c2, g3)
    mean_dist = msum[0, 0] / _B
    return (zq.reshape(_B, _D), idx.reshape(_B), mean_dist)


# speedup vs baseline: 2.1170x; 2.1170x over previous
import functools

import jax
import jax.numpy as jnp
from jax.experimental import pallas as pl
from jax.experimental.pallas import tpu as pltpu

_B = 1048576
_D = 16
_K = 64
_BB = 8192
_R = _BB // 8
_GCOL = 136


def _vq_block(x_ref, ct2_ref, c2_ref, g3_ref, zq_ref, idx_ref, msum_ref):
    x = x_ref[...]
    e = jax.lax.dot_general(x, ct2_ref[...], (((1,), (0,)), ((), ())),
                            preferred_element_type=jnp.float32)
    e = e + c2_ref[...]
    m = jnp.minimum(e[:, :256], e[:, 256:])
    m = jnp.minimum(m[:, :128], m[:, 128:])
    for s in (8, 16, 32, 64):
        m = jnp.minimum(m, pltpu.roll(m, s, 1))
    mbig = jnp.concatenate([m, m, m, m], axis=1)
    onehot = (e == mbig).astype(jnp.float32)
    out2 = jax.lax.dot_general(onehot, g3_ref[...], (((1,), (0,)), ((), ())),
                               preferred_element_type=jnp.float32)
    zq_ref[...] = out2[:, :128]
    idx_ref[...] = out2[:, 128:_GCOL].astype(jnp.int32)
    s_val = jnp.sum(x * x) + jnp.sum(m) * (1.0 / 16.0)

    @pl.when(pl.program_id(0) == 0)
    def _init():
        msum_ref[0, 0] = 0.0

    msum_ref[0, 0] += s_val


def _build_tables(codebook):
    eye8 = jnp.eye(8, dtype=jnp.float32)
    ct2 = jnp.einsum('gh,kd->gdkh', eye8, -2.0 * codebook).reshape(128, 8 * _K)
    c2k = jnp.sum(codebook * codebook, axis=1)
    c2 = jnp.broadcast_to(c2k[:, None], (_K, 8)).reshape(1, 8 * _K)
    zpart = jnp.einsum('hg,kd->khgd', eye8, codebook).reshape(8 * _K, 128)
    kf = jnp.arange(_K, dtype=jnp.float32)
    ipart = jnp.einsum('k,hg->khg', kf, eye8).reshape(8 * _K, 8)
    g3 = jnp.concatenate([zpart, ipart], axis=1)
    return ct2, c2, g3


def kernel(action, codebook):
    xv = action.reshape(_B // 8, 128)
    ct2, c2, g3 = _build_tables(codebook)
    n_blocks = _B // _BB
    zq, idx, msum = pl.pallas_call(
        _vq_block,
        grid=(n_blocks,),
        in_specs=[
            pl.BlockSpec((_R, 128), lambda i: (i, 0)),
            pl.BlockSpec((128, 8 * _K), lambda i: (0, 0)),
            pl.BlockSpec((1, 8 * _K), lambda i: (0, 0)),
            pl.BlockSpec((8 * _K, _GCOL), lambda i: (0, 0)),
        ],
        out_specs=[
            pl.BlockSpec((_R, 128), lambda i: (i, 0)),
            pl.BlockSpec((_R, 8), lambda i: (i, 0)),
            pl.BlockSpec((1, 1), lambda i: (0, 0), memory_space=pltpu.SMEM),
        ],
        out_shape=[
            jax.ShapeDtypeStruct((_B // 8, 128), jnp.float32),
            jax.ShapeDtypeStruct((_B // 8, 8), jnp.int32),
            jax.ShapeDtypeStruct((1, 1), jnp.float32),
        ],
    )(xv, ct2, c2, g3)
    mean_dist = msum[0, 0] / _B
    return (zq.reshape(_B, _D), idx.reshape(_B), mean_dist)


# speedup vs baseline: 16.2829x; 7.6915x over previous
import functools

import jax
import jax.numpy as jnp
from jax.experimental import pallas as pl
from jax.experimental.pallas import tpu as pltpu

_B = 1048576
_D = 16
_K = 64
_BB = 8192


def _vq_block(at_ref, ct_ref, c2_ref, g_ref, zqt_ref, idx_ref, msum_ref):
    at = at_ref[...]
    e = jax.lax.dot_general(ct_ref[...], at, (((1,), (0,)), ((), ())),
                            preferred_element_type=jnp.float32)
    e = e + c2_ref[...]
    m = jnp.min(e, axis=0, keepdims=True)
    onehot = (e == m).astype(jnp.float32)
    out2 = jax.lax.dot_general(g_ref[...], onehot, (((1,), (0,)), ((), ())),
                               preferred_element_type=jnp.float32)
    zqt_ref[...] = out2[:_D, :]
    idx_ref[...] = out2[_D:_D + 1, :].astype(jnp.int32)
    s_val = jnp.sum(at * at) + jnp.sum(m)

    @pl.when(pl.program_id(0) == 0)
    def _init():
        msum_ref[0, 0] = 0.0

    msum_ref[0, 0] += s_val


def kernel(action, codebook):
    at = action.T
    ct = -2.0 * codebook
    c2 = jnp.sum(codebook * codebook, axis=1)[:, None]
    kf = jnp.arange(_K, dtype=jnp.float32)[None, :]
    g = jnp.concatenate([codebook.T, kf], axis=0)
    n_blocks = _B // _BB
    zqt, idx, msum = pl.pallas_call(
        _vq_block,
        grid=(n_blocks,),
        in_specs=[
            pl.BlockSpec((_D, _BB), lambda i: (0, i)),
            pl.BlockSpec((_K, _D), lambda i: (0, 0)),
            pl.BlockSpec((_K, 1), lambda i: (0, 0)),
            pl.BlockSpec((_D + 1, _K), lambda i: (0, 0)),
        ],
        out_specs=[
            pl.BlockSpec((_D, _BB), lambda i: (0, i)),
            pl.BlockSpec((1, _BB), lambda i: (0, i)),
            pl.BlockSpec((1, 1), lambda i: (0, 0), memory_space=pltpu.SMEM),
        ],
        out_shape=[
            jax.ShapeDtypeStruct((_D, _B), jnp.float32),
            jax.ShapeDtypeStruct((1, _B), jnp.int32),
            jax.ShapeDtypeStruct((1, 1), jnp.float32),
        ],
    )(at, ct, c2, g)
    mean_dist = msum[0, 0] / _B
    return (zqt.T, idx.reshape(_B), mean_dist)


# speedup vs baseline: 27.1608x; 1.6681x over previous
import functools

import jax
import jax.numpy as jnp
from jax.experimental import pallas as pl
from jax.experimental.pallas import tpu as pltpu

_B = 1048576
_D = 16
_K = 64
_BB = 32768


def _vq_block(at_ref, ct_ref, c2_ref, g_ref, zqt_ref, idx_ref, msum_ref):
    at = at_ref[...]
    e = jax.lax.dot_general(ct_ref[...], at, (((1,), (0,)), ((), ())),
                            preferred_element_type=jnp.float32)
    e = e + c2_ref[...]
    m = jnp.min(e, axis=0, keepdims=True)
    onehot = (e == m).astype(jnp.float32)
    out2 = jax.lax.dot_general(g_ref[...], onehot, (((1,), (0,)), ((), ())),
                               preferred_element_type=jnp.float32)
    zqt_ref[...] = out2[:_D, :]
    idx_ref[...] = out2[_D:_D + 1, :].astype(jnp.int32)
    s_val = jnp.sum(at * at) + jnp.sum(m)

    @pl.when(pl.program_id(0) == 0)
    def _init():
        msum_ref[0, 0] = 0.0

    msum_ref[0, 0] += s_val


def kernel(action, codebook):
    at = action.T
    ct = -2.0 * codebook
    c2 = jnp.sum(codebook * codebook, axis=1)[:, None]
    kf = jnp.arange(_K, dtype=jnp.float32)[None, :]
    g = jnp.concatenate([codebook.T, kf], axis=0)
    n_blocks = _B // _BB
    zqt, idx, msum = pl.pallas_call(
        _vq_block,
        grid=(n_blocks,),
        in_specs=[
            pl.BlockSpec((_D, _BB), lambda i: (0, i)),
            pl.BlockSpec((_K, _D), lambda i: (0, 0)),
            pl.BlockSpec((_K, 1), lambda i: (0, 0)),
            pl.BlockSpec((_D + 1, _K), lambda i: (0, 0)),
        ],
        out_specs=[
            pl.BlockSpec((_D, _BB), lambda i: (0, i)),
            pl.BlockSpec((1, _BB), lambda i: (0, i)),
            pl.BlockSpec((1, 1), lambda i: (0, 0), memory_space=pltpu.SMEM),
        ],
        out_shape=[
            jax.ShapeDtypeStruct((_D, _B), jnp.float32),
            jax.ShapeDtypeStruct((1, _B), jnp.int32),
            jax.ShapeDtypeStruct((1, 1), jnp.float32),
        ],
    )(at, ct, c2, g)
    mean_dist = msum[0, 0] / _B
    return (zqt.T, idx.reshape(_B), mean_dist)


# speedup vs baseline: 30.1858x; 1.1114x over previous
import functools

import jax
import jax.numpy as jnp
from jax.experimental import pallas as pl
from jax.experimental.pallas import tpu as pltpu

_B = 1048576
_D = 16
_K = 64
_BB = 65536


def _vq_block(at_ref, ct_ref, c2_ref, g_ref, zqt_ref, idx_ref, msum_ref):
    at = at_ref[...]
    e = jax.lax.dot_general(ct_ref[...], at, (((1,), (0,)), ((), ())),
                            preferred_element_type=jnp.float32)
    e = e + c2_ref[...]
    m = jnp.min(e, axis=0, keepdims=True)
    onehot = (e == m).astype(jnp.float32)
    out2 = jax.lax.dot_general(g_ref[...], onehot, (((1,), (0,)), ((), ())),
                               preferred_element_type=jnp.float32)
    zqt_ref[...] = out2[:_D, :]
    idx_ref[...] = out2[_D:_D + 1, :].astype(jnp.int32)
    s_val = jnp.sum(at * at) + jnp.sum(m)

    @pl.when(pl.program_id(0) == 0)
    def _init():
        msum_ref[0, 0] = 0.0

    msum_ref[0, 0] += s_val


def kernel(action, codebook):
    at = action.T
    ct = -2.0 * codebook
    c2 = jnp.sum(codebook * codebook, axis=1)[:, None]
    kf = jnp.arange(_K, dtype=jnp.float32)[None, :]
    g = jnp.concatenate([codebook.T, kf], axis=0)
    n_blocks = _B // _BB
    zqt, idx, msum = pl.pallas_call(
        _vq_block,
        grid=(n_blocks,),
        in_specs=[
            pl.BlockSpec((_D, _BB), lambda i: (0, i)),
            pl.BlockSpec((_K, _D), lambda i: (0, 0)),
            pl.BlockSpec((_K, 1), lambda i: (0, 0)),
            pl.BlockSpec((_D + 1, _K), lambda i: (0, 0)),
        ],
        out_specs=[
            pl.BlockSpec((_D, _BB), lambda i: (0, i)),
            pl.BlockSpec((1, _BB), lambda i: (0, i)),
            pl.BlockSpec((1, 1), lambda i: (0, 0), memory_space=pltpu.SMEM),
        ],
        out_shape=[
            jax.ShapeDtypeStruct((_D, _B), jnp.float32),
            jax.ShapeDtypeStruct((1, _B), jnp.int32),
            jax.ShapeDtypeStruct((1, 1), jnp.float32),
        ],
    )(at, ct, c2, g)
    mean_dist = msum[0, 0] / _B
    return (zqt.T, idx.reshape(_B), mean_dist)
